# baseline (speedup 1.0000x reference)
import jax
import jax.numpy as jnp
from jax.experimental import pallas as pl
from jax.experimental.pallas import tpu as pltpu

_B, _H, _L, _D, _S = 8, 16, 2048, 128, 16
_BH = _B * _H
_BH_PER_CHUNK = 2
_CHROWS = _BH_PER_CHUNK * _L
_NTASK_PER_CACHE = _BH // _BH_PER_CHUNK
_NBUF = 6


def _body(pos_ref, kc, vc, kv, vv, ko, vo, valbuf, sem_val, *rest):
    bufs = rest[:_NBUF]
    sem_r = rest[_NBUF:2 * _NBUF]
    sem_w = rest[2 * _NBUF:3 * _NBUF]

    tasks = [(c, t) for c in range(2) for t in range(_NTASK_PER_CACHE)]
    T = len(tasks)
    srcs = (kc, vc)
    dsts = (ko, vo)

    cp_kv = pltpu.make_async_copy(kv, valbuf.at[0], sem_val)
    cp_vv = pltpu.make_async_copy(vv, valbuf.at[1], sem_val)
    cp_kv.start()
    cp_vv.start()

    def rows(t):
        _, chunk = tasks[t]
        return pl.ds(chunk * _CHROWS, _CHROWS)

    def start_read(t, slot):
        c, _ = tasks[t]
        cp = pltpu.make_async_copy(srcs[c].at[rows(t)], bufs[slot], sem_r[slot])
        cp.start()
        return cp

    def start_write(t, slot):
        c, _ = tasks[t]
        cp = pltpu.make_async_copy(bufs[slot], dsts[c].at[rows(t)], sem_w[slot])
        cp.start()
        return cp

    def patch(t, slot):
        c, chunk = tasks[t]
        for j in range(_BH_PER_CHUNK):
            bh = chunk * _BH_PER_CHUNK + j
            for i in range(_S):
                p = pos_ref[i]
                bufs[slot][pl.ds(j * _L + p, 1), :] = (
                    valbuf[c, pl.ds(bh * _S + i, 1), :])

    reads = [None] * T
    writes = [None] * T
    for t in range(min(_NBUF, T)):
        reads[t] = start_read(t, t % _NBUF)
    cp_kv.wait()
    cp_vv.wait()
    for t in range(T):
        slot = t % _NBUF
        reads[t].wait()
        patch(t, slot)
        writes[t] = start_write(t, slot)
        u = t - (_NBUF - 1)
        if u >= 0 and u + _NBUF < T:
            writes[u].wait()
            reads[u + _NBUF] = start_read(u + _NBUF, u % _NBUF)
    for t in range(max(0, T - _NBUF), T):
        writes[t].wait()


def kernel(input_pos, k_val, v_val, k_cache, v_cache):
    kc = k_cache.reshape(_BH * _L, _D)
    vc = v_cache.reshape(_BH * _L, _D)
    kv = k_val.reshape(_BH * _S, _D)
    vv = v_val.reshape(_BH * _S, _D)

    any_spec = pl.BlockSpec(memory_space=pl.ANY)
    ko, vo = pl.pallas_call(
        _body,
        in_specs=[
            pl.BlockSpec(memory_space=pltpu.SMEM),
            any_spec, any_spec, any_spec, any_spec,
        ],
        out_specs=[any_spec, any_spec],
        out_shape=[
            jax.ShapeDtypeStruct((_BH * _L, _D), jnp.float32),
            jax.ShapeDtypeStruct((_BH * _L, _D), jnp.float32),
        ],
        scratch_shapes=(
            [pltpu.VMEM((2, _BH * _S, _D), jnp.float32),
             pltpu.SemaphoreType.DMA]
            + [pltpu.VMEM((_CHROWS, _D), jnp.float32) for _ in range(_NBUF)]
            + [pltpu.SemaphoreType.DMA for _ in range(2 * _NBUF)]
        ),
    )(input_pos, kc, vc, kv, vv)
    return (ko.reshape(_B, _H, _L, _D), vo.reshape(_B, _H, _L, _D))


# speedup vs baseline: 1.8072x; 1.8072x over previous
import jax
import jax.numpy as jnp
from jax.experimental import pallas as pl
from jax.experimental.pallas import tpu as pltpu

_B, _H, _L, _D, _S = 8, 16, 2048, 128, 16
_GB = 2


def _body(pos_ref, kc_ref, vc_ref, kv_ref, vv_ref, ko_ref, vo_ref):
    ko_ref[...] = kc_ref[...]
    vo_ref[...] = vc_ref[...]
    for i in range(_S):
        p = pos_ref[i]
        for j in range(_GB):
            ko_ref[j, pl.ds(p, 1), :] = kv_ref[j, pl.ds(i, 1), :]
            vo_ref[j, pl.ds(p, 1), :] = vv_ref[j, pl.ds(i, 1), :]


def kernel(input_pos, k_val, v_val, k_cache, v_cache):
    bh = _B * _H
    kc = k_cache.reshape(bh, _L, _D)
    vc = v_cache.reshape(bh, _L, _D)
    kv = k_val.reshape(bh, _S, _D)
    vv = v_val.reshape(bh, _S, _D)

    cache_spec = pl.BlockSpec((_GB, _L, _D), lambda i: (i, 0, 0))
    val_spec = pl.BlockSpec((_GB, _S, _D), lambda i: (i, 0, 0))
    ko, vo = pl.pallas_call(
        _body,
        grid=(bh // _GB,),
        in_specs=[
            pl.BlockSpec(memory_space=pltpu.SMEM),
            cache_spec,
            cache_spec,
            val_spec,
            val_spec,
        ],
        out_specs=[cache_spec, cache_spec],
        out_shape=[
            jax.ShapeDtypeStruct((bh, _L, _D), jnp.float32),
            jax.ShapeDtypeStruct((bh, _L, _D), jnp.float32),
        ],
        compiler_params=pltpu.CompilerParams(
            dimension_semantics=("arbitrary",),
        ),
    )(input_pos, kc, vc, kv, vv)
    return (ko.reshape(_B, _H, _L, _D), vo.reshape(_B, _H, _L, _D))


# speedup vs baseline: 1.8317x; 1.0135x over previous
import jax
import jax.numpy as jnp
from jax.experimental import pallas as pl
from jax.experimental.pallas import tpu as pltpu

_B, _H, _L, _D, _S = 8, 16, 2048, 128, 16
_GB = 8


def _body(pos_ref, c_ref, v_ref, o_ref):
    o_ref[...] = c_ref[...]
    for i in range(_S):
        p = pos_ref[i]
        for j in range(_GB):
            o_ref[j, pl.ds(p, 1), :] = v_ref[j, pl.ds(i, 1), :]


def _update(input_pos, val, cache):
    bh = _B * _H
    cache_spec = pl.BlockSpec((_GB, _L, _D), lambda i: (i, 0, 0))
    val_spec = pl.BlockSpec((_GB, _S, _D), lambda i: (i, 0, 0))
    return pl.pallas_call(
        _body,
        grid=(bh // _GB,),
        in_specs=[
            pl.BlockSpec(memory_space=pltpu.SMEM),
            cache_spec,
            val_spec,
        ],
        out_specs=cache_spec,
        out_shape=jax.ShapeDtypeStruct((bh, _L, _D), jnp.float32),
        compiler_params=pltpu.CompilerParams(
            dimension_semantics=("arbitrary",),
        ),
    )(input_pos, cache, val)


def kernel(input_pos, k_val, v_val, k_cache, v_cache):
    bh = _B * _H
    ko = _update(input_pos, k_val.reshape(bh, _S, _D),
                 k_cache.reshape(bh, _L, _D))
    vo = _update(input_pos, v_val.reshape(bh, _S, _D),
                 v_cache.reshape(bh, _L, _D))
    return (ko.reshape(_B, _H, _L, _D), vo.reshape(_B, _H, _L, _D))
